# baseline (speedup 1.0000x reference)
import functools

import jax
import jax.numpy as jnp
from jax import lax
from jax.experimental import pallas as pl
from jax.experimental.pallas import tpu as pltpu
from jax.experimental.pallas import tpu_sc as plsc

B, L, D = 4, 8192, 768
NC, NS, LANES = 2, 16, 16
NW = NC * NS
ROWS_W = L // NW
T = 16
STEPS = ROWS_W // T


class _Both:
    def __init__(self, a, b):
        self.a, self.b = a, b

    def wait(self):
        self.a.wait()
        self.b.wait()


def _sc_body(x_hbm, t_hbm, o_hbm, t0, t1, x0, x1,
             s_t0, s_t1, s_xi0, s_xi1, s_xo0, s_xo1):
    t_bufs, x_bufs = (t0, t1), (x0, x1)
    s_t, s_xi, s_xo = (s_t0, s_t1), (s_xi0, s_xi1), (s_xo0, s_xo1)

    wid = lax.axis_index("s") * NC + lax.axis_index("c")
    row_at = lambda s: wid * ROWS_W + s * T

    def add_half(xq, t_v, r0):
        @plsc.parallel_loop(r0, r0 + T // 2)
        def _(r):
            @plsc.parallel_loop(0, D, step=LANES, unroll=8)
            def _(c):
                tv = t_v[r, pl.ds(c, LANES)]
                for bb in range(B):
                    plsc.addupdate(xq.at[bb, r, pl.ds(c, LANES)], tv)

    tin = [None, None]
    xin = [None, None]
    xout = [None, None]
    tin[0] = pltpu.async_copy(t_hbm.at[pl.ds(row_at(0), T)], t_bufs[0], s_t[0])
    xin[0] = pltpu.async_copy(x_hbm.at[:, pl.ds(row_at(0), T)], x_bufs[0],
                              s_xi[0])

    for s in range(STEPS):
        cur, nxt = s % 2, (s + 1) % 2
        if s + 1 < STEPS:
            if xout[nxt] is not None:
                xout[nxt].wait()
            xin[nxt] = pltpu.async_copy(
                x_hbm.at[:, pl.ds(row_at(s + 1), T)], x_bufs[nxt], s_xi[nxt])
            tin[nxt] = pltpu.async_copy(
                t_hbm.at[pl.ds(row_at(s + 1), T)], t_bufs[nxt], s_t[nxt])
        tin[cur].wait()
        xin[cur].wait()
        add_half(x_bufs[cur], t_bufs[cur], 0)
        xouth = pltpu.async_copy(
            x_bufs[cur].at[:, pl.ds(0, T // 2)],
            o_hbm.at[:, pl.ds(row_at(s), T // 2)], s_xo[cur])
        add_half(x_bufs[cur], t_bufs[cur], T // 2)
        xout[cur] = pltpu.async_copy(
            x_bufs[cur].at[:, pl.ds(T // 2, T // 2)],
            o_hbm.at[:, pl.ds(row_at(s) + T // 2, T // 2)], s_xo[cur])
        xout[cur] = _Both(xouth, xout[cur])

    xout[(STEPS - 1) % 2].wait()
    xout[STEPS % 2].wait()


@functools.partial(
    pl.kernel,
    out_type=jax.ShapeDtypeStruct((B, L, D), jnp.float32),
    mesh=plsc.VectorSubcoreMesh(core_axis_name="c", subcore_axis_name="s"),
    scratch_types=[
        pltpu.VMEM((T, D), jnp.float32),
        pltpu.VMEM((T, D), jnp.float32),
        pltpu.VMEM((B, T, D), jnp.float32),
        pltpu.VMEM((B, T, D), jnp.float32),
        pltpu.SemaphoreType.DMA,
        pltpu.SemaphoreType.DMA,
        pltpu.SemaphoreType.DMA,
        pltpu.SemaphoreType.DMA,
        pltpu.SemaphoreType.DMA,
        pltpu.SemaphoreType.DMA,
    ],
)
def _sc_add(*refs):
    _sc_body(*refs)


def kernel(x, table):
    return _sc_add(x, table)


# speedup vs baseline: 1.0238x; 1.0238x over previous
import functools

import jax
import jax.numpy as jnp
from jax import lax
from jax.experimental import pallas as pl
from jax.experimental.pallas import tpu as pltpu
from jax.experimental.pallas import tpu_sc as plsc

B, L, D = 4, 8192, 768
NC, NS, LANES = 2, 16, 16
NW = NC * NS
ROWS_W = L // NW
T = 16
STEPS = ROWS_W // T


def _sc_body(x_hbm, t_hbm, o_hbm, t0, t1, x0, x1,
             s_t0, s_t1, s_xi0, s_xi1, s_xo0, s_xo1):
    t_bufs, x_bufs = (t0, t1), (x0, x1)
    s_t, s_xi, s_xo = (s_t0, s_t1), (s_xi0, s_xi1), (s_xo0, s_xo1)

    wid = lax.axis_index("s") * NC + lax.axis_index("c")
    row_at = lambda s: wid * ROWS_W + s * T

    def add_quad(xq, t_v):
        @plsc.parallel_loop(0, T)
        def _(r):
            @plsc.parallel_loop(0, D, step=LANES, unroll=8)
            def _(c):
                tv = t_v[r, pl.ds(c, LANES)]
                for bb in range(B):
                    plsc.addupdate(xq.at[bb, r, pl.ds(c, LANES)], tv)

    tin = [None, None]
    xin = [None, None]
    xout = [None, None]
    tin[0] = pltpu.async_copy(t_hbm.at[pl.ds(row_at(0), T)], t_bufs[0], s_t[0])
    xin[0] = pltpu.async_copy(x_hbm.at[:, pl.ds(row_at(0), T)], x_bufs[0],
                              s_xi[0])

    for s in range(STEPS):
        cur, nxt = s % 2, (s + 1) % 2
        if s + 1 < STEPS:
            if xout[nxt] is not None:
                xout[nxt].wait()
            xin[nxt] = pltpu.async_copy(
                x_hbm.at[:, pl.ds(row_at(s + 1), T)], x_bufs[nxt], s_xi[nxt])
            tin[nxt] = pltpu.async_copy(
                t_hbm.at[pl.ds(row_at(s + 1), T)], t_bufs[nxt], s_t[nxt])
        tin[cur].wait()
        xin[cur].wait()
        add_quad(x_bufs[cur], t_bufs[cur])
        xout[cur] = pltpu.async_copy(
            x_bufs[cur], o_hbm.at[:, pl.ds(row_at(s), T)], s_xo[cur])

    xout[(STEPS - 1) % 2].wait()
    xout[STEPS % 2].wait()


@functools.partial(
    pl.kernel,
    out_type=jax.ShapeDtypeStruct((B, L, D), jnp.float32),
    mesh=plsc.VectorSubcoreMesh(core_axis_name="c", subcore_axis_name="s"),
    scratch_types=[
        pltpu.VMEM((T, D), jnp.float32),
        pltpu.VMEM((T, D), jnp.float32),
        pltpu.VMEM((B, T, D), jnp.float32),
        pltpu.VMEM((B, T, D), jnp.float32),
        pltpu.SemaphoreType.DMA,
        pltpu.SemaphoreType.DMA,
        pltpu.SemaphoreType.DMA,
        pltpu.SemaphoreType.DMA,
        pltpu.SemaphoreType.DMA,
        pltpu.SemaphoreType.DMA,
    ],
)
def _sc_add(*refs):
    _sc_body(*refs)


def kernel(x, table):
    return _sc_add(x, table)


# speedup vs baseline: 1.0412x; 1.0170x over previous
import functools

import jax
import jax.numpy as jnp
from jax import lax
from jax.experimental import pallas as pl
from jax.experimental.pallas import tpu as pltpu
from jax.experimental.pallas import tpu_sc as plsc

B, L, D = 4, 8192, 768
NC, NS, LANES = 2, 16, 16
NW = NC * NS
ROWS_W = L // NW
T = 8
STEPS = ROWS_W // T
NB = 3


def _sc_body(x_hbm, t_hbm, o_hbm, t0, t1, t2, x0, x1, x2,
             s_t0, s_t1, s_t2, s_xi0, s_xi1, s_xi2, s_xo0, s_xo1, s_xo2):
    t_bufs, x_bufs = (t0, t1, t2), (x0, x1, x2)
    s_t, s_xi, s_xo = (s_t0, s_t1, s_t2), (s_xi0, s_xi1, s_xi2), (s_xo0,
                                                                  s_xo1,
                                                                  s_xo2)

    wid = lax.axis_index("s") * NC + lax.axis_index("c")
    row_at = lambda s: wid * ROWS_W + s * T

    def add_quad(xq, t_v):
        @plsc.parallel_loop(0, T)
        def _(r):
            @plsc.parallel_loop(0, D, step=LANES, unroll=4)
            def _(c):
                tv = t_v[r, pl.ds(c, LANES)]
                for bb in range(B):
                    plsc.addupdate(xq.at[bb, r, pl.ds(c, LANES)], tv)

    tin = [None] * NB
    xin = [None] * NB
    xout = [None] * NB
    for p in range(NB - 1):
        tin[p] = pltpu.async_copy(t_hbm.at[pl.ds(row_at(p), T)], t_bufs[p],
                                  s_t[p])
        xin[p] = pltpu.async_copy(x_hbm.at[:, pl.ds(row_at(p), T)], x_bufs[p],
                                  s_xi[p])

    for s in range(STEPS):
        cur = s % NB
        if s + NB - 1 < STEPS:
            nxt = (s + NB - 1) % NB
            if xout[nxt] is not None:
                xout[nxt].wait()
            xin[nxt] = pltpu.async_copy(
                x_hbm.at[:, pl.ds(row_at(s + NB - 1), T)], x_bufs[nxt],
                s_xi[nxt])
            tin[nxt] = pltpu.async_copy(
                t_hbm.at[pl.ds(row_at(s + NB - 1), T)], t_bufs[nxt],
                s_t[nxt])
        tin[cur].wait()
        xin[cur].wait()
        add_quad(x_bufs[cur], t_bufs[cur])
        xout[cur] = pltpu.async_copy(
            x_bufs[cur], o_hbm.at[:, pl.ds(row_at(s), T)], s_xo[cur])

    for k in range(min(NB, STEPS)):
        xout[(STEPS - 1 - k) % NB].wait()


@functools.partial(
    pl.kernel,
    out_type=jax.ShapeDtypeStruct((B, L, D), jnp.float32),
    mesh=plsc.VectorSubcoreMesh(core_axis_name="c", subcore_axis_name="s"),
    scratch_types=[
        pltpu.VMEM((T, D), jnp.float32),
        pltpu.VMEM((T, D), jnp.float32),
        pltpu.VMEM((T, D), jnp.float32),
        pltpu.VMEM((B, T, D), jnp.float32),
        pltpu.VMEM((B, T, D), jnp.float32),
        pltpu.VMEM((B, T, D), jnp.float32),
        pltpu.SemaphoreType.DMA,
        pltpu.SemaphoreType.DMA,
        pltpu.SemaphoreType.DMA,
        pltpu.SemaphoreType.DMA,
        pltpu.SemaphoreType.DMA,
        pltpu.SemaphoreType.DMA,
        pltpu.SemaphoreType.DMA,
        pltpu.SemaphoreType.DMA,
        pltpu.SemaphoreType.DMA,
    ],
)
def _sc_add(*refs):
    _sc_body(*refs)


def kernel(x, table):
    return _sc_add(x, table)
